# Initial kernel scaffold; baseline (speedup 1.0000x reference)
import jax
import jax.numpy as jnp
from jax.experimental import pallas as pl


def kernel(x, edge_index, edge_attr, r_iter, W_node_in, W_edge, W_f1, W_f2, W_f3, W_ge1, W_ge2, W_gn1, W_gn2):
    raise NotImplementedError("write your pallas kernel here")


# speedup vs baseline: 1.0051x; 1.0051x over previous
import jax
import jax.numpy as jnp
from jax.experimental import pallas as pl
from jax.experimental.pallas import tpu as pltpu

_ML = 128
_ZETA = 1e-6
_ILAYERS = 2
_NITER = 2
_BE = 4000

_SELU_ALPHA = 1.6732632423543772
_SELU_SCALE = 1.0507009873554805


def _selu(v):
  return _SELU_SCALE * jnp.where(v > 0, v, _SELU_ALPHA * (jnp.exp(v) - 1.0))


def _edge1_body(qc_ref, w_ref, wq_ref, w2_ref, out_ref):
  zc = jnp.dot(_selu(qc_ref[...]), wq_ref[...],
               preferred_element_type=jnp.float32)
  m = _selu(w_ref[...] + zc)
  out_ref[...] = jnp.dot(m, w2_ref[...], preferred_element_type=jnp.float32)


def _edge2_body(z_ref, w_ref, c_ref, w2_ref, out_ref):
  m = _selu(w_ref[...] + jnp.dot(z_ref[...], c_ref[...],
                                 preferred_element_type=jnp.float32))
  out_ref[...] = jnp.dot(m, w2_ref[...], preferred_element_type=jnp.float32)


def _edgef_body(z_ref, w_ref, cf_ref, w2_ref, w3t_ref, out_ref):
  a = w_ref[...] + jnp.dot(_selu(z_ref[...]), cf_ref[...],
                           preferred_element_type=jnp.float32)
  b = _selu(jnp.dot(_selu(a), w2_ref[...], preferred_element_type=jnp.float32))
  out_ref[...] = jnp.sum(b * w3t_ref[...], axis=1, keepdims=True)


def _full(shape):
  return pl.BlockSpec(shape, lambda i: (0, 0))


def _run_edge1(qc, w, wq, w2):
  e = qc.shape[0]
  return pl.pallas_call(
      _edge1_body,
      grid=(e // _BE,),
      in_specs=[
          pl.BlockSpec((_BE, 2), lambda i: (i, 0)),
          pl.BlockSpec((_BE, _ML), lambda i: (i, 0)),
          _full((2, _ML)),
          _full((_ML, _ML)),
      ],
      out_specs=pl.BlockSpec((_BE, _ML), lambda i: (i, 0)),
      out_shape=jax.ShapeDtypeStruct((e, _ML), jnp.float32),
      compiler_params=pltpu.CompilerParams(
          dimension_semantics=("parallel",)),
  )(qc, w, wq, w2)


def _run_edge2(z, w, c, w2):
  e = z.shape[0]
  return pl.pallas_call(
      _edge2_body,
      grid=(e // _BE,),
      in_specs=[
          pl.BlockSpec((_BE, _ML), lambda i: (i, 0)),
          pl.BlockSpec((_BE, _ML), lambda i: (i, 0)),
          _full((_ML, _ML)),
          _full((_ML, _ML)),
      ],
      out_specs=pl.BlockSpec((_BE, _ML), lambda i: (i, 0)),
      out_shape=jax.ShapeDtypeStruct((e, _ML), jnp.float32),
      compiler_params=pltpu.CompilerParams(
          dimension_semantics=("parallel",)),
  )(z, w, c, w2)


def _run_edgef(z, w, cf, w2, w3t):
  e = z.shape[0]
  return pl.pallas_call(
      _edgef_body,
      grid=(e // _BE,),
      in_specs=[
          pl.BlockSpec((_BE, _ML), lambda i: (i, 0)),
          pl.BlockSpec((_BE, _ML), lambda i: (i, 0)),
          _full((_ML, _ML)),
          _full((_ML, _ML)),
          pl.BlockSpec((1, _ML), lambda i: (0, 0)),
      ],
      out_specs=pl.BlockSpec((_BE, 1), lambda i: (i, 0)),
      out_shape=jax.ShapeDtypeStruct((e, 1), jnp.float32),
      compiler_params=pltpu.CompilerParams(
          dimension_semantics=("parallel",)),
  )(z, w, cf, w2, w3t)


def _net_flows(h, r, src, dst, n_nodes):
  dh = h[src] - h[dst]
  q = jnp.sign(dh) * ((jnp.abs(dh) + _ZETA) / (r + _ZETA)) ** 0.54
  d = jax.ops.segment_sum(q, dst, num_segments=n_nodes)
  return d, q


def _heads(j_steps, h_star, mask, q, r, src, dst, n_nodes):
  hl = r * jnp.sign(q) * (jnp.abs(q) + _ZETA) ** 1.852
  h = h_star
  for _ in range(j_steps):
    cand = h[src] - hl
    m = jax.ops.segment_max(cand, dst, num_segments=n_nodes)
    m = jnp.where(jnp.isfinite(m), m, h)
    h = jnp.where(mask, h_star, m)
  return h


def kernel(x, edge_index, edge_attr, r_iter, W_node_in, W_edge, W_f1, W_f2,
           W_f3, W_ge1, W_ge2, W_gn1, W_gn2):
  src = edge_index[0].astype(jnp.int32)
  dst = edge_index[1].astype(jnp.int32)
  r = edge_attr[:, 0:1]
  d_star = x[:, 1:2]
  h_star = x[:, 0:1]
  mask = h_star != 0
  n_nodes = x.shape[0]
  e = src.shape[0]
  half = e // 2

  a1, b1, c1 = W_ge1[0][:_ML], W_ge1[0][_ML:2 * _ML], W_ge1[0][2 * _ML:]
  a2, b2, c2 = W_ge1[1][:_ML], W_ge1[1][_ML:2 * _ML], W_ge1[1][2 * _ML:]
  fa, fb, fc = W_f1[:_ML], W_f1[_ML:2 * _ML], W_f1[2 * _ML:]
  wq1 = W_edge @ c1
  ga1, gb1 = W_gn1[0][:_ML], W_gn1[0][_ML:]
  ga2, gb2 = W_gn1[1][:_ML], W_gn1[1][_ML:]
  w3t = W_f3.T

  d_hat, q_hat = _net_flows(h_star, r, src, dst, n_nodes)
  q_tilde = q_hat
  h_tilde = h_star
  j_steps = _ILAYERS * _NITER
  k_total = _NITER + r_iter

  def _outer(_, carry):
    d_hat, q_hat, q_tilde, h_tilde = carry
    g = _selu(jnp.concatenate([d_hat, d_star], axis=-1)) @ W_node_in
    w = (g @ a1)[src] + (g @ b1)[dst]
    qc = jnp.concatenate([q_tilde, q_hat], axis=-1)
    z = _run_edge1(qc, w, wq1, W_ge2[0])
    aggr = jax.ops.segment_max(z, dst, num_segments=n_nodes)
    aggr = jnp.where(jnp.isfinite(aggr), aggr, 0.0)
    g = _selu(g @ ga1 + aggr @ gb1) @ W_gn2[0]
    w = (g @ a2)[src] + (g @ b2)[dst]
    z = _run_edge2(z, w, c2, W_ge2[1])
    aggr = jax.ops.segment_max(z, dst, num_segments=n_nodes)
    aggr = jnp.where(jnp.isfinite(aggr), aggr, 0.0)
    g = _selu(g @ ga2 + aggr @ gb2) @ W_gn2[1]
    sg = _selu(g)
    w = (sg @ fa)[src] + (sg @ fb)[dst]
    dq = _run_edgef(z, w, fc, W_f2, w3t)
    q_hat = q_hat + dq
    q_in = q_hat[:half]
    q_hat = jnp.concatenate([q_in, -q_in], axis=0)
    d_hat = jax.ops.segment_sum(q_hat, dst, num_segments=n_nodes)
    h_tilde = _heads(j_steps, h_star, mask, q_hat, r, src, dst, n_nodes)
    _, q_tilde = _net_flows(h_tilde, r, src, dst, n_nodes)
    return (d_hat, q_hat, q_tilde, h_tilde)

  _, _, _, h_tilde = jax.lax.fori_loop(
      0, k_total, _outer, (d_hat, q_hat, q_tilde, h_tilde))
  return h_tilde


# speedup vs baseline: 1.1844x; 1.1784x over previous
import functools

import jax
import jax.numpy as jnp
from jax import lax
from jax.experimental import pallas as pl
from jax.experimental.pallas import tpu as pltpu
from jax.experimental.pallas import tpu_sc as plsc

_ML = 128
_ZETA = 1e-6
_ILAYERS = 2
_NITER = 2
_BE = 4000

_SELU_ALPHA = 1.6732632423543772
_SELU_SCALE = 1.0507009873554805


def _selu(v):
  return _SELU_SCALE * jnp.where(v > 0, v, _SELU_ALPHA * (jnp.exp(v) - 1.0))


_SC_NC = 2
_SC_NS = 16
_SC_NW = _SC_NC * _SC_NS
_SC_CH = 80


def _sc_gather_pair(u, v, src_idx, dst_idx):
  e = src_idx.shape[0]
  epw = e // _SC_NW

  @functools.partial(
      pl.kernel,
      out_type=[jax.ShapeDtypeStruct((e, _ML), jnp.float32),
                jax.ShapeDtypeStruct((e, _ML), jnp.float32)],
      scratch_types=[
          pltpu.VMEM((epw,), jnp.int32),
          pltpu.VMEM((epw,), jnp.int32),
          pltpu.VMEM((_SC_CH, _ML), jnp.float32),
          pltpu.VMEM((_SC_CH, _ML), jnp.float32),
          pltpu.SemaphoreType.DMA,
          pltpu.SemaphoreType.DMA,
      ],
      mesh=plsc.VectorSubcoreMesh(core_axis_name="c", subcore_axis_name="s"),
  )
  def run(u_hbm, v_hbm, si_hbm, di_hbm, us_hbm, vs_hbm,
          idxs_v, idxd_v, urows_v, vrows_v, sem_u, sem_v):
    wid = lax.axis_index("s") * _SC_NC + lax.axis_index("c")
    base = wid * epw
    pltpu.sync_copy(si_hbm.at[pl.ds(base, epw)], idxs_v)
    pltpu.sync_copy(di_hbm.at[pl.ds(base, epw)], idxd_v)

    def chunk(c, carry):
      off = c * _SC_CH
      cp_u = pltpu.async_copy(
          u_hbm.at[idxs_v.at[pl.ds(off, _SC_CH)]], urows_v, sem_u)
      cp_v = pltpu.async_copy(
          v_hbm.at[idxd_v.at[pl.ds(off, _SC_CH)]], vrows_v, sem_v)
      cp_u.wait()
      cp_v.wait()
      pltpu.sync_copy(urows_v, us_hbm.at[pl.ds(base + off, _SC_CH)])
      pltpu.sync_copy(vrows_v, vs_hbm.at[pl.ds(base + off, _SC_CH)])
      return carry

    lax.fori_loop(0, epw // _SC_CH, chunk, 0)

  return run(u, v, src_idx, dst_idx)


def _edge1_body(qc_ref, us_ref, vs_ref, wq_ref, w2_ref, out_ref):
  zc = jnp.dot(_selu(qc_ref[...]), wq_ref[...],
               preferred_element_type=jnp.float32)
  m = _selu(us_ref[...] + vs_ref[...] + zc)
  out_ref[...] = jnp.dot(m, w2_ref[...], preferred_element_type=jnp.float32)


def _edge2_body(z_ref, us_ref, vs_ref, c_ref, w2_ref, out_ref):
  m = _selu(us_ref[...] + vs_ref[...] +
            jnp.dot(z_ref[...], c_ref[...],
                    preferred_element_type=jnp.float32))
  out_ref[...] = jnp.dot(m, w2_ref[...], preferred_element_type=jnp.float32)


def _edgef_body(z_ref, us_ref, vs_ref, cf_ref, w2_ref, w3t_ref, out_ref):
  a = us_ref[...] + vs_ref[...] + jnp.dot(_selu(z_ref[...]), cf_ref[...],
                                          preferred_element_type=jnp.float32)
  b = _selu(jnp.dot(_selu(a), w2_ref[...], preferred_element_type=jnp.float32))
  out_ref[...] = jnp.sum(b * w3t_ref[...], axis=1, keepdims=True)


def _full(shape):
  return pl.BlockSpec(shape, lambda i: (0, 0))


def _edge_block(shape=( _BE, _ML)):
  return pl.BlockSpec(shape, lambda i: (i, 0))


def _run_edge1(qc, us, vs, wq, w2):
  e = qc.shape[0]
  return pl.pallas_call(
      _edge1_body,
      grid=(e // _BE,),
      in_specs=[
          _edge_block((_BE, 2)), _edge_block(), _edge_block(),
          _full((2, _ML)), _full((_ML, _ML)),
      ],
      out_specs=_edge_block(),
      out_shape=jax.ShapeDtypeStruct((e, _ML), jnp.float32),
      compiler_params=pltpu.CompilerParams(
          dimension_semantics=("parallel",)),
  )(qc, us, vs, wq, w2)


def _run_edge2(z, us, vs, c, w2):
  e = z.shape[0]
  return pl.pallas_call(
      _edge2_body,
      grid=(e // _BE,),
      in_specs=[
          _edge_block(), _edge_block(), _edge_block(),
          _full((_ML, _ML)), _full((_ML, _ML)),
      ],
      out_specs=_edge_block(),
      out_shape=jax.ShapeDtypeStruct((e, _ML), jnp.float32),
      compiler_params=pltpu.CompilerParams(
          dimension_semantics=("parallel",)),
  )(z, us, vs, c, w2)


def _run_edgef(z, us, vs, cf, w2, w3t):
  e = z.shape[0]
  return pl.pallas_call(
      _edgef_body,
      grid=(e // _BE,),
      in_specs=[
          _edge_block(), _edge_block(), _edge_block(),
          _full((_ML, _ML)), _full((_ML, _ML)),
          pl.BlockSpec((1, _ML), lambda i: (0, 0)),
      ],
      out_specs=_edge_block((_BE, 1)),
      out_shape=jax.ShapeDtypeStruct((e, 1), jnp.float32),
      compiler_params=pltpu.CompilerParams(
          dimension_semantics=("parallel",)),
  )(z, us, vs, cf, w2, w3t)


def _net_flows(h, r, src, dst, n_nodes):
  dh = h[src] - h[dst]
  q = jnp.sign(dh) * ((jnp.abs(dh) + _ZETA) / (r + _ZETA)) ** 0.54
  d = jax.ops.segment_sum(q, dst, num_segments=n_nodes)
  return d, q


def _heads(j_steps, h_star, mask, q, r, src, dst, n_nodes):
  hl = r * jnp.sign(q) * (jnp.abs(q) + _ZETA) ** 1.852
  h = h_star
  for _ in range(j_steps):
    cand = h[src] - hl
    m = jax.ops.segment_max(cand, dst, num_segments=n_nodes)
    m = jnp.where(jnp.isfinite(m), m, h)
    h = jnp.where(mask, h_star, m)
  return h


def kernel(x, edge_index, edge_attr, r_iter, W_node_in, W_edge, W_f1, W_f2,
           W_f3, W_ge1, W_ge2, W_gn1, W_gn2):
  src = edge_index[0].astype(jnp.int32)
  dst = edge_index[1].astype(jnp.int32)
  r = edge_attr[:, 0:1]
  d_star = x[:, 1:2]
  h_star = x[:, 0:1]
  mask = h_star != 0
  n_nodes = x.shape[0]
  e = src.shape[0]
  half = e // 2

  a1, b1, c1 = W_ge1[0][:_ML], W_ge1[0][_ML:2 * _ML], W_ge1[0][2 * _ML:]
  a2, b2, c2 = W_ge1[1][:_ML], W_ge1[1][_ML:2 * _ML], W_ge1[1][2 * _ML:]
  fa, fb, fc = W_f1[:_ML], W_f1[_ML:2 * _ML], W_f1[2 * _ML:]
  wq1 = W_edge @ c1
  ga1, gb1 = W_gn1[0][:_ML], W_gn1[0][_ML:]
  ga2, gb2 = W_gn1[1][:_ML], W_gn1[1][_ML:]
  w3t = W_f3.T

  d_hat, q_hat = _net_flows(h_star, r, src, dst, n_nodes)
  q_tilde = q_hat
  h_tilde = h_star
  j_steps = _ILAYERS * _NITER
  k_total = _NITER + r_iter

  def _outer(_, carry):
    d_hat, q_hat, q_tilde, h_tilde = carry
    g = _selu(jnp.concatenate([d_hat, d_star], axis=-1)) @ W_node_in
    us, vs = _sc_gather_pair(g @ a1, g @ b1, src, dst)
    qc = jnp.concatenate([q_tilde, q_hat], axis=-1)
    z = _run_edge1(qc, us, vs, wq1, W_ge2[0])
    aggr = jax.ops.segment_max(z, dst, num_segments=n_nodes)
    aggr = jnp.where(jnp.isfinite(aggr), aggr, 0.0)
    g = _selu(g @ ga1 + aggr @ gb1) @ W_gn2[0]
    us, vs = _sc_gather_pair(g @ a2, g @ b2, src, dst)
    z = _run_edge2(z, us, vs, c2, W_ge2[1])
    aggr = jax.ops.segment_max(z, dst, num_segments=n_nodes)
    aggr = jnp.where(jnp.isfinite(aggr), aggr, 0.0)
    g = _selu(g @ ga2 + aggr @ gb2) @ W_gn2[1]
    sg = _selu(g)
    us, vs = _sc_gather_pair(sg @ fa, sg @ fb, src, dst)
    dq = _run_edgef(z, us, vs, fc, W_f2, w3t)
    q_hat = q_hat + dq
    q_in = q_hat[:half]
    q_hat = jnp.concatenate([q_in, -q_in], axis=0)
    d_hat = jax.ops.segment_sum(q_hat, dst, num_segments=n_nodes)
    h_tilde = _heads(j_steps, h_star, mask, q_hat, r, src, dst, n_nodes)
    _, q_tilde = _net_flows(h_tilde, r, src, dst, n_nodes)
    return (d_hat, q_hat, q_tilde, h_tilde)

  _, _, _, h_tilde = jax.lax.fori_loop(
      0, k_total, _outer, (d_hat, q_hat, q_tilde, h_tilde))
  return h_tilde


# speedup vs baseline: 3.0676x; 2.5900x over previous
import functools

import jax
import jax.numpy as jnp
from jax import lax
from jax.experimental import pallas as pl
from jax.experimental.pallas import tpu as pltpu
from jax.experimental.pallas import tpu_sc as plsc

_ML = 128
_ZETA = 1e-6
_ILAYERS = 2
_NITER = 2
_BE = 4000

_SELU_ALPHA = 1.6732632423543772
_SELU_SCALE = 1.0507009873554805


def _selu(v):
  return _SELU_SCALE * jnp.where(v > 0, v, _SELU_ALPHA * (jnp.exp(v) - 1.0))


_SC_NC = 2
_SC_NS = 16
_SC_NW = _SC_NC * _SC_NS
_SC_CH = 80


def _sc_gather_pair(u, v, src_idx, dst_idx):
  e = src_idx.shape[0]
  epw = e // _SC_NW

  @functools.partial(
      pl.kernel,
      out_type=[jax.ShapeDtypeStruct((e, _ML), jnp.float32),
                jax.ShapeDtypeStruct((e, _ML), jnp.float32)],
      scratch_types=[
          pltpu.VMEM((epw,), jnp.int32),
          pltpu.VMEM((epw,), jnp.int32),
          pltpu.VMEM((_SC_CH, _ML), jnp.float32),
          pltpu.VMEM((_SC_CH, _ML), jnp.float32),
          pltpu.SemaphoreType.DMA,
          pltpu.SemaphoreType.DMA,
      ],
      mesh=plsc.VectorSubcoreMesh(core_axis_name="c", subcore_axis_name="s"),
  )
  def run(u_hbm, v_hbm, si_hbm, di_hbm, us_hbm, vs_hbm,
          idxs_v, idxd_v, urows_v, vrows_v, sem_u, sem_v):
    wid = lax.axis_index("s") * _SC_NC + lax.axis_index("c")
    base = wid * epw
    pltpu.sync_copy(si_hbm.at[pl.ds(base, epw)], idxs_v)
    pltpu.sync_copy(di_hbm.at[pl.ds(base, epw)], idxd_v)

    def chunk(c, carry):
      off = c * _SC_CH
      cp_u = pltpu.async_copy(
          u_hbm.at[idxs_v.at[pl.ds(off, _SC_CH)]], urows_v, sem_u)
      cp_v = pltpu.async_copy(
          v_hbm.at[idxd_v.at[pl.ds(off, _SC_CH)]], vrows_v, sem_v)
      cp_u.wait()
      cp_v.wait()
      pltpu.sync_copy(urows_v, us_hbm.at[pl.ds(base + off, _SC_CH)])
      pltpu.sync_copy(vrows_v, vs_hbm.at[pl.ds(base + off, _SC_CH)])
      return carry

    lax.fori_loop(0, epw // _SC_CH, chunk, 0)

  return run(u, v, src_idx, dst_idx)


_SC_SCH = 80
_SC_SK = 5


def _sc_take_scalar(tab, idx3):
  nch = idx3.shape[1]

  @functools.partial(
      pl.kernel,
      out_type=jax.ShapeDtypeStruct((_SC_NW, nch, _SC_SCH), jnp.float32),
      scratch_types=[
          pltpu.VMEM((nch, _SC_SCH), jnp.int32),
          pltpu.VMEM((nch, _SC_SCH), jnp.float32),
          pltpu.SemaphoreType.DMA,
      ],
      mesh=plsc.VectorSubcoreMesh(core_axis_name="c", subcore_axis_name="s"),
  )
  def run(tab_hbm, idx_hbm, out_hbm, idx_v, out_v, sem):
    wid = lax.axis_index("s") * _SC_NC + lax.axis_index("c")
    pltpu.sync_copy(idx_hbm.at[wid], idx_v)

    def chunk(c, carry):
      row = c * _SC_SK
      cps = [pltpu.async_copy(tab_hbm.at[idx_v.at[row + j]],
                              out_v.at[row + j], sem)
             for j in range(_SC_SK)]
      for cp in cps:
        cp.wait()
      return carry

    lax.fori_loop(0, nch // _SC_SK, chunk, 0)
    pltpu.sync_copy(out_v, out_hbm.at[wid])

  return run(tab, idx3)


def _edge1_body(qc_ref, us_ref, vs_ref, wq_ref, w2_ref, out_ref):
  zc = jnp.dot(_selu(qc_ref[...]), wq_ref[...],
               preferred_element_type=jnp.float32)
  m = _selu(us_ref[...] + vs_ref[...] + zc)
  out_ref[...] = jnp.dot(m, w2_ref[...], preferred_element_type=jnp.float32)


def _edge2_body(z_ref, us_ref, vs_ref, c_ref, w2_ref, out_ref):
  m = _selu(us_ref[...] + vs_ref[...] +
            jnp.dot(z_ref[...], c_ref[...],
                    preferred_element_type=jnp.float32))
  out_ref[...] = jnp.dot(m, w2_ref[...], preferred_element_type=jnp.float32)


def _edgef_body(z_ref, us_ref, vs_ref, cf_ref, w2_ref, w3t_ref, out_ref):
  a = us_ref[...] + vs_ref[...] + jnp.dot(_selu(z_ref[...]), cf_ref[...],
                                          preferred_element_type=jnp.float32)
  b = _selu(jnp.dot(_selu(a), w2_ref[...], preferred_element_type=jnp.float32))
  out_ref[...] = jnp.sum(b * w3t_ref[...], axis=1, keepdims=True)


def _full(shape):
  return pl.BlockSpec(shape, lambda i: (0, 0))


def _edge_block(shape=( _BE, _ML)):
  return pl.BlockSpec(shape, lambda i: (i, 0))


def _run_edge1(qc, us, vs, wq, w2):
  e = qc.shape[0]
  return pl.pallas_call(
      _edge1_body,
      grid=(e // _BE,),
      in_specs=[
          _edge_block((_BE, 2)), _edge_block(), _edge_block(),
          _full((2, _ML)), _full((_ML, _ML)),
      ],
      out_specs=_edge_block(),
      out_shape=jax.ShapeDtypeStruct((e, _ML), jnp.float32),
      compiler_params=pltpu.CompilerParams(
          dimension_semantics=("parallel",)),
  )(qc, us, vs, wq, w2)


def _run_edge2(z, us, vs, c, w2):
  e = z.shape[0]
  return pl.pallas_call(
      _edge2_body,
      grid=(e // _BE,),
      in_specs=[
          _edge_block(), _edge_block(), _edge_block(),
          _full((_ML, _ML)), _full((_ML, _ML)),
      ],
      out_specs=_edge_block(),
      out_shape=jax.ShapeDtypeStruct((e, _ML), jnp.float32),
      compiler_params=pltpu.CompilerParams(
          dimension_semantics=("parallel",)),
  )(z, us, vs, c, w2)


def _run_edgef(z, us, vs, cf, w2, w3t):
  e = z.shape[0]
  return pl.pallas_call(
      _edgef_body,
      grid=(e // _BE,),
      in_specs=[
          _edge_block(), _edge_block(), _edge_block(),
          _full((_ML, _ML)), _full((_ML, _ML)),
          pl.BlockSpec((1, _ML), lambda i: (0, 0)),
      ],
      out_specs=_edge_block((_BE, 1)),
      out_shape=jax.ShapeDtypeStruct((e, 1), jnp.float32),
      compiler_params=pltpu.CompilerParams(
          dimension_semantics=("parallel",)),
  )(z, us, vs, cf, w2, w3t)


def _net_flows_init(h, r, src, dst, n_nodes):
  dh = h[src] - h[dst]
  q = jnp.sign(dh) * ((jnp.abs(dh) + _ZETA) / (r + _ZETA)) ** 0.54
  d = jax.ops.segment_sum(q, dst, num_segments=n_nodes)
  return d, q


def _net_flows_sc(h, r, src3, dst3, dst, n_nodes):
  e = dst.shape[0]
  hs = _sc_take_scalar(h[:, 0], src3).reshape(e, 1)
  hd = _sc_take_scalar(h[:, 0], dst3).reshape(e, 1)
  dh = hs - hd
  q = jnp.sign(dh) * ((jnp.abs(dh) + _ZETA) / (r + _ZETA)) ** 0.54
  d = jax.ops.segment_sum(q, dst, num_segments=n_nodes)
  return d, q


def _heads_sc(j_steps, h_star, mask, q, r, src3, dst, n_nodes):
  e = dst.shape[0]
  hl = r * jnp.sign(q) * (jnp.abs(q) + _ZETA) ** 1.852
  h = h_star
  for _ in range(j_steps):
    cand = _sc_take_scalar(h[:, 0], src3).reshape(e, 1) - hl
    m = jax.ops.segment_max(cand, dst, num_segments=n_nodes)
    m = jnp.where(jnp.isfinite(m), m, h)
    h = jnp.where(mask, h_star, m)
  return h


def kernel(x, edge_index, edge_attr, r_iter, W_node_in, W_edge, W_f1, W_f2,
           W_f3, W_ge1, W_ge2, W_gn1, W_gn2):
  src = edge_index[0].astype(jnp.int32)
  dst = edge_index[1].astype(jnp.int32)
  r = edge_attr[:, 0:1]
  d_star = x[:, 1:2]
  h_star = x[:, 0:1]
  mask = h_star != 0
  n_nodes = x.shape[0]
  e = src.shape[0]
  half = e // 2

  a1, b1, c1 = W_ge1[0][:_ML], W_ge1[0][_ML:2 * _ML], W_ge1[0][2 * _ML:]
  a2, b2, c2 = W_ge1[1][:_ML], W_ge1[1][_ML:2 * _ML], W_ge1[1][2 * _ML:]
  fa, fb, fc = W_f1[:_ML], W_f1[_ML:2 * _ML], W_f1[2 * _ML:]
  wq1 = W_edge @ c1
  ga1, gb1 = W_gn1[0][:_ML], W_gn1[0][_ML:]
  ga2, gb2 = W_gn1[1][:_ML], W_gn1[1][_ML:]
  w3t = W_f3.T

  nch = e // _SC_NW // _SC_SCH
  src3 = src.reshape(_SC_NW, nch, _SC_SCH)
  dst3 = dst.reshape(_SC_NW, nch, _SC_SCH)

  d_hat, q_hat = _net_flows_init(h_star, r, src, dst, n_nodes)
  q_tilde = q_hat
  h_tilde = h_star
  j_steps = _ILAYERS * _NITER
  k_total = _NITER + r_iter

  def _outer(_, carry):
    d_hat, q_hat, q_tilde, h_tilde = carry
    g = _selu(jnp.concatenate([d_hat, d_star], axis=-1)) @ W_node_in
    us, vs = _sc_gather_pair(g @ a1, g @ b1, src, dst)
    qc = jnp.concatenate([q_tilde, q_hat], axis=-1)
    z = _run_edge1(qc, us, vs, wq1, W_ge2[0])
    aggr = jax.ops.segment_max(z, dst, num_segments=n_nodes)
    aggr = jnp.where(jnp.isfinite(aggr), aggr, 0.0)
    g = _selu(g @ ga1 + aggr @ gb1) @ W_gn2[0]
    us, vs = _sc_gather_pair(g @ a2, g @ b2, src, dst)
    z = _run_edge2(z, us, vs, c2, W_ge2[1])
    aggr = jax.ops.segment_max(z, dst, num_segments=n_nodes)
    aggr = jnp.where(jnp.isfinite(aggr), aggr, 0.0)
    g = _selu(g @ ga2 + aggr @ gb2) @ W_gn2[1]
    sg = _selu(g)
    us, vs = _sc_gather_pair(sg @ fa, sg @ fb, src, dst)
    dq = _run_edgef(z, us, vs, fc, W_f2, w3t)
    q_hat = q_hat + dq
    q_in = q_hat[:half]
    q_hat = jnp.concatenate([q_in, -q_in], axis=0)
    d_hat = jax.ops.segment_sum(q_hat, dst, num_segments=n_nodes)
    h_tilde = _heads_sc(j_steps, h_star, mask, q_hat, r, src3, dst, n_nodes)
    _, q_tilde = _net_flows_sc(h_tilde, r, src3, dst3, dst, n_nodes)
    return (d_hat, q_hat, q_tilde, h_tilde)

  _, _, _, h_tilde = jax.lax.fori_loop(
      0, k_total, _outer, (d_hat, q_hat, q_tilde, h_tilde))
  return h_tilde


# speedup vs baseline: 3.8506x; 1.2552x over previous
import functools

import jax
import jax.numpy as jnp
from jax import lax
from jax.experimental import pallas as pl
from jax.experimental.pallas import tpu as pltpu
from jax.experimental.pallas import tpu_sc as plsc

_ML = 128
_ZETA = 1e-6
_ILAYERS = 2
_NITER = 2
_BE = 4000

_SELU_ALPHA = 1.6732632423543772
_SELU_SCALE = 1.0507009873554805


def _selu(v):
  return _SELU_SCALE * jnp.where(v > 0, v, _SELU_ALPHA * (jnp.exp(v) - 1.0))


_SC_NC = 2
_SC_NS = 16
_SC_NW = _SC_NC * _SC_NS
_SC_CH = 80


def _sc_gather_pair(u, v, src_idx, dst_idx):
  e = src_idx.shape[0]
  epw = e // _SC_NW

  @functools.partial(
      pl.kernel,
      out_type=[jax.ShapeDtypeStruct((e, _ML), jnp.float32),
                jax.ShapeDtypeStruct((e, _ML), jnp.float32)],
      scratch_types=[
          pltpu.VMEM((epw,), jnp.int32),
          pltpu.VMEM((epw,), jnp.int32),
          pltpu.VMEM((_SC_CH, _ML), jnp.float32),
          pltpu.VMEM((_SC_CH, _ML), jnp.float32),
          pltpu.SemaphoreType.DMA,
          pltpu.SemaphoreType.DMA,
      ],
      mesh=plsc.VectorSubcoreMesh(core_axis_name="c", subcore_axis_name="s"),
  )
  def run(u_hbm, v_hbm, si_hbm, di_hbm, us_hbm, vs_hbm,
          idxs_v, idxd_v, urows_v, vrows_v, sem_u, sem_v):
    wid = lax.axis_index("s") * _SC_NC + lax.axis_index("c")
    base = wid * epw
    pltpu.sync_copy(si_hbm.at[pl.ds(base, epw)], idxs_v)
    pltpu.sync_copy(di_hbm.at[pl.ds(base, epw)], idxd_v)

    def chunk(c, carry):
      off = c * _SC_CH
      cp_u = pltpu.async_copy(
          u_hbm.at[idxs_v.at[pl.ds(off, _SC_CH)]], urows_v, sem_u)
      cp_v = pltpu.async_copy(
          v_hbm.at[idxd_v.at[pl.ds(off, _SC_CH)]], vrows_v, sem_v)
      cp_u.wait()
      cp_v.wait()
      pltpu.sync_copy(urows_v, us_hbm.at[pl.ds(base + off, _SC_CH)])
      pltpu.sync_copy(vrows_v, vs_hbm.at[pl.ds(base + off, _SC_CH)])
      return carry

    lax.fori_loop(0, epw // _SC_CH, chunk, 0)

  return run(u, v, src_idx, dst_idx)


_SC_SCH = 80
_SC_SK = 5


def _sc_take_scalar(tab, idx3):
  nch = idx3.shape[1]

  @functools.partial(
      pl.kernel,
      out_type=jax.ShapeDtypeStruct((_SC_NW, nch, _SC_SCH), jnp.float32),
      scratch_types=[
          pltpu.VMEM((nch, _SC_SCH), jnp.int32),
          pltpu.VMEM((nch, _SC_SCH), jnp.float32),
          pltpu.SemaphoreType.DMA,
      ],
      mesh=plsc.VectorSubcoreMesh(core_axis_name="c", subcore_axis_name="s"),
  )
  def run(tab_hbm, idx_hbm, out_hbm, idx_v, out_v, sem):
    wid = lax.axis_index("s") * _SC_NC + lax.axis_index("c")
    pltpu.sync_copy(idx_hbm.at[wid], idx_v)

    def chunk(c, carry):
      row = c * _SC_SK
      cps = [pltpu.async_copy(tab_hbm.at[idx_v.at[row + j]],
                              out_v.at[row + j], sem)
             for j in range(_SC_SK)]
      for cp in cps:
        cp.wait()
      return carry

    lax.fori_loop(0, nch // _SC_SK, chunk, 0)
    pltpu.sync_copy(out_v, out_hbm.at[wid])

  return run(tab, idx3)


def _sc_segsum_scalar(vals, dst3, n_nodes):
  nch = dst3.shape[1]
  vals3 = vals.reshape(_SC_NW, nch, _SC_SCH)

  @functools.partial(
      pl.kernel,
      out_type=jax.ShapeDtypeStruct((_SC_NC, n_nodes), jnp.float32),
      scratch_types=[
          pltpu.VMEM((nch, _SC_SCH), jnp.float32),
          pltpu.VMEM((nch, _SC_SCH), jnp.int32),
          pltpu.VMEM((n_nodes,), jnp.float32),
          pltpu.VMEM_SHARED((n_nodes,), jnp.float32),
      ],
      mesh=plsc.VectorSubcoreMesh(core_axis_name="c", subcore_axis_name="s"),
  )
  def run(vals_hbm, idx_hbm, out_hbm, vals_v, idx_v, zbuf_v, acc_sh):
    sid = lax.axis_index("s")
    cid = lax.axis_index("c")
    wid = sid * _SC_NC + cid
    pltpu.sync_copy(vals_hbm.at[wid], vals_v)
    pltpu.sync_copy(idx_hbm.at[wid], idx_v)

    @pl.when(sid == 0)
    def _zero():
      def zb(i, carry):
        zbuf_v[pl.ds(i * 16, 16)] = jnp.zeros((16,), jnp.float32)
        return carry
      lax.fori_loop(0, n_nodes // 16, zb, 0)
      pltpu.sync_copy(zbuf_v, acc_sh)

    plsc.subcore_barrier()

    def chunk(c, carry):
      pltpu.sync_copy(vals_v.at[c], acc_sh.at[idx_v.at[c]], add=True)
      return carry

    lax.fori_loop(0, nch, chunk, 0)
    plsc.subcore_barrier()

    @pl.when(sid == 0)
    def _out():
      pltpu.sync_copy(acc_sh, out_hbm.at[cid])

  parts = run(vals3, dst3)
  return (parts[0] + parts[1]).reshape(n_nodes, 1)


def _edge1_body(qc_ref, us_ref, vs_ref, wq_ref, w2_ref, out_ref):
  zc = jnp.dot(_selu(qc_ref[...]), wq_ref[...],
               preferred_element_type=jnp.float32)
  m = _selu(us_ref[...] + vs_ref[...] + zc)
  out_ref[...] = jnp.dot(m, w2_ref[...], preferred_element_type=jnp.float32)


def _edge2_body(z_ref, us_ref, vs_ref, c_ref, w2_ref, out_ref):
  m = _selu(us_ref[...] + vs_ref[...] +
            jnp.dot(z_ref[...], c_ref[...],
                    preferred_element_type=jnp.float32))
  out_ref[...] = jnp.dot(m, w2_ref[...], preferred_element_type=jnp.float32)


def _edgef_body(z_ref, us_ref, vs_ref, cf_ref, w2_ref, w3t_ref, out_ref):
  a = us_ref[...] + vs_ref[...] + jnp.dot(_selu(z_ref[...]), cf_ref[...],
                                          preferred_element_type=jnp.float32)
  b = _selu(jnp.dot(_selu(a), w2_ref[...], preferred_element_type=jnp.float32))
  out_ref[...] = jnp.sum(b * w3t_ref[...], axis=1, keepdims=True)


def _full(shape):
  return pl.BlockSpec(shape, lambda i: (0, 0))


def _edge_block(shape=( _BE, _ML)):
  return pl.BlockSpec(shape, lambda i: (i, 0))


def _run_edge1(qc, us, vs, wq, w2):
  e = qc.shape[0]
  return pl.pallas_call(
      _edge1_body,
      grid=(e // _BE,),
      in_specs=[
          _edge_block((_BE, 2)), _edge_block(), _edge_block(),
          _full((2, _ML)), _full((_ML, _ML)),
      ],
      out_specs=_edge_block(),
      out_shape=jax.ShapeDtypeStruct((e, _ML), jnp.float32),
      compiler_params=pltpu.CompilerParams(
          dimension_semantics=("parallel",)),
  )(qc, us, vs, wq, w2)


def _run_edge2(z, us, vs, c, w2):
  e = z.shape[0]
  return pl.pallas_call(
      _edge2_body,
      grid=(e // _BE,),
      in_specs=[
          _edge_block(), _edge_block(), _edge_block(),
          _full((_ML, _ML)), _full((_ML, _ML)),
      ],
      out_specs=_edge_block(),
      out_shape=jax.ShapeDtypeStruct((e, _ML), jnp.float32),
      compiler_params=pltpu.CompilerParams(
          dimension_semantics=("parallel",)),
  )(z, us, vs, c, w2)


def _run_edgef(z, us, vs, cf, w2, w3t):
  e = z.shape[0]
  return pl.pallas_call(
      _edgef_body,
      grid=(e // _BE,),
      in_specs=[
          _edge_block(), _edge_block(), _edge_block(),
          _full((_ML, _ML)), _full((_ML, _ML)),
          pl.BlockSpec((1, _ML), lambda i: (0, 0)),
      ],
      out_specs=_edge_block((_BE, 1)),
      out_shape=jax.ShapeDtypeStruct((e, 1), jnp.float32),
      compiler_params=pltpu.CompilerParams(
          dimension_semantics=("parallel",)),
  )(z, us, vs, cf, w2, w3t)


def _net_flows_sc(h, r, src3, dst3, e):
  hs = _sc_take_scalar(h[:, 0], src3).reshape(e, 1)
  hd = _sc_take_scalar(h[:, 0], dst3).reshape(e, 1)
  dh = hs - hd
  return jnp.sign(dh) * ((jnp.abs(dh) + _ZETA) / (r + _ZETA)) ** 0.54


def _heads_sc(j_steps, h_star, mask, q, r, src3, dst, n_nodes):
  e = dst.shape[0]
  hl = r * jnp.sign(q) * (jnp.abs(q) + _ZETA) ** 1.852
  h = h_star
  for _ in range(j_steps):
    cand = _sc_take_scalar(h[:, 0], src3).reshape(e, 1) - hl
    m = jax.ops.segment_max(cand, dst, num_segments=n_nodes)
    m = jnp.where(jnp.isfinite(m), m, h)
    h = jnp.where(mask, h_star, m)
  return h


def kernel(x, edge_index, edge_attr, r_iter, W_node_in, W_edge, W_f1, W_f2,
           W_f3, W_ge1, W_ge2, W_gn1, W_gn2):
  src = edge_index[0].astype(jnp.int32)
  dst = edge_index[1].astype(jnp.int32)
  r = edge_attr[:, 0:1]
  d_star = x[:, 1:2]
  h_star = x[:, 0:1]
  mask = h_star != 0
  n_nodes = x.shape[0]
  e = src.shape[0]
  half = e // 2

  a1, b1, c1 = W_ge1[0][:_ML], W_ge1[0][_ML:2 * _ML], W_ge1[0][2 * _ML:]
  a2, b2, c2 = W_ge1[1][:_ML], W_ge1[1][_ML:2 * _ML], W_ge1[1][2 * _ML:]
  fa, fb, fc = W_f1[:_ML], W_f1[_ML:2 * _ML], W_f1[2 * _ML:]
  wq1 = W_edge @ c1
  ga1, gb1 = W_gn1[0][:_ML], W_gn1[0][_ML:]
  ga2, gb2 = W_gn1[1][:_ML], W_gn1[1][_ML:]
  w3t = W_f3.T

  nch = e // _SC_NW // _SC_SCH
  src3 = src.reshape(_SC_NW, nch, _SC_SCH)
  dst3 = dst.reshape(_SC_NW, nch, _SC_SCH)

  q_hat = _net_flows_sc(h_star, r, src3, dst3, e)
  d_hat = _sc_segsum_scalar(q_hat[:, 0], dst3, n_nodes)
  q_tilde = q_hat
  h_tilde = h_star
  j_steps = _ILAYERS * _NITER
  k_total = _NITER + r_iter

  def _outer(_, carry):
    d_hat, q_hat, q_tilde, h_tilde = carry
    g = _selu(jnp.concatenate([d_hat, d_star], axis=-1)) @ W_node_in
    us, vs = _sc_gather_pair(g @ a1, g @ b1, src, dst)
    qc = jnp.concatenate([q_tilde, q_hat], axis=-1)
    z = _run_edge1(qc, us, vs, wq1, W_ge2[0])
    aggr = jax.ops.segment_max(z, dst, num_segments=n_nodes)
    aggr = jnp.where(jnp.isfinite(aggr), aggr, 0.0)
    g = _selu(g @ ga1 + aggr @ gb1) @ W_gn2[0]
    us, vs = _sc_gather_pair(g @ a2, g @ b2, src, dst)
    z = _run_edge2(z, us, vs, c2, W_ge2[1])
    aggr = jax.ops.segment_max(z, dst, num_segments=n_nodes)
    aggr = jnp.where(jnp.isfinite(aggr), aggr, 0.0)
    g = _selu(g @ ga2 + aggr @ gb2) @ W_gn2[1]
    sg = _selu(g)
    us, vs = _sc_gather_pair(sg @ fa, sg @ fb, src, dst)
    dq = _run_edgef(z, us, vs, fc, W_f2, w3t)
    q_hat = q_hat + dq
    q_in = q_hat[:half]
    q_hat = jnp.concatenate([q_in, -q_in], axis=0)
    d_hat = _sc_segsum_scalar(q_hat[:, 0], dst3, n_nodes)
    h_tilde = _heads_sc(j_steps, h_star, mask, q_hat, r, src3, dst, n_nodes)
    q_tilde = _net_flows_sc(h_tilde, r, src3, dst3, e)
    return (d_hat, q_hat, q_tilde, h_tilde)

  _, _, _, h_tilde = jax.lax.fori_loop(
      0, k_total, _outer, (d_hat, q_hat, q_tilde, h_tilde))
  return h_tilde
